# baseline (device time: 107875 ns/iter reference)
import jax
import jax.numpy as jnp
from jax import lax
from jax.experimental import pallas as pl
from jax.experimental.pallas import tpu as pltpu

N_DEV = 4
SQ = 1024
SKV = 1024
HQ_PER = 8
DH = 128
D_MODEL = 1024
SCALE = 0.08838834764831843


def kernel(x, Wq, K_ext, V_ext, Wo):
    my_pos = lax.axis_index("i")

    xb = x[0].astype(jnp.bfloat16)
    wq_sl = lax.dynamic_slice(
        Wq.astype(jnp.bfloat16), (0, my_pos * HQ_PER * DH), (D_MODEL, HQ_PER * DH)
    )
    wo_sl = lax.dynamic_slice(
        Wo.astype(jnp.bfloat16), (my_pos * HQ_PER * DH, 0), (HQ_PER * DH, D_MODEL)
    )
    k = K_ext[0].transpose(1, 0, 2).astype(jnp.bfloat16)
    v = V_ext[0].transpose(1, 0, 2).astype(jnp.bfloat16)

    def body(x_ref, wq_ref, k_ref, v_ref, wo_ref, out_ref,
             comm_ref, send_sems, recv_sems):
        my = lax.axis_index("i")
        left = lax.rem(my + N_DEV - 1, N_DEV)
        right = lax.rem(my + 1, N_DEV)

        q = jnp.dot(x_ref[...], wq_ref[...],
                    preferred_element_type=jnp.float32).astype(jnp.bfloat16)

        rows = lax.broadcasted_iota(jnp.int32, (SQ, SKV), 0)
        cols = lax.broadcasted_iota(jnp.int32, (SQ, SKV), 1)
        mask = ((rows // 64) % 4) == ((cols // 64) % 4)

        ctx_parts = []
        for h in range(HQ_PER):
            q_h = q[:, h * DH:(h + 1) * DH]
            s = lax.dot_general(
                q_h, k_ref[h], (((1,), (1,)), ((), ())),
                preferred_element_type=jnp.float32,
            ) * SCALE
            s = jnp.where(mask, s, -1e9)
            m = jnp.max(s, axis=-1, keepdims=True)
            w = jnp.exp(s - m)
            w = w / jnp.sum(w, axis=-1, keepdims=True)
            ctx_parts.append(
                jnp.dot(w.astype(jnp.bfloat16), v_ref[h],
                        preferred_element_type=jnp.float32).astype(jnp.bfloat16)
            )
        ctx = jnp.concatenate(ctx_parts, axis=1)

        partial = jnp.dot(ctx, wo_ref[...],
                          preferred_element_type=jnp.float32)
        out_ref[...] = partial
        comm_ref[0, :, :] = partial.astype(jnp.bfloat16)

        barrier_sem = pltpu.get_barrier_semaphore()
        for nbr in [left, right]:
            pl.semaphore_signal(
                barrier_sem, inc=1,
                device_id=(nbr,), device_id_type=pl.DeviceIdType.MESH,
            )
        pl.semaphore_wait(barrier_sem, 2)

        for h in range(N_DEV - 1):
            send_slot = h % 2
            recv_slot = (h + 1) % 2
            rdma = pltpu.make_async_remote_copy(
                src_ref=comm_ref.at[send_slot],
                dst_ref=comm_ref.at[recv_slot],
                send_sem=send_sems.at[send_slot],
                recv_sem=recv_sems.at[recv_slot],
                device_id=(right,),
                device_id_type=pl.DeviceIdType.MESH,
            )
            rdma.start()
            rdma.wait()
            out_ref[...] += comm_ref[recv_slot].astype(jnp.float32)

    out = pl.pallas_call(
        body,
        out_shape=jax.ShapeDtypeStruct((SQ, D_MODEL), jnp.float32),
        in_specs=[pl.BlockSpec(memory_space=pltpu.VMEM)] * 5,
        out_specs=pl.BlockSpec(memory_space=pltpu.VMEM),
        scratch_shapes=[
            pltpu.VMEM((2, SQ, D_MODEL), jnp.bfloat16),
            pltpu.SemaphoreType.DMA((2,)),
            pltpu.SemaphoreType.DMA((2,)),
        ],
        compiler_params=pltpu.CompilerParams(collective_id=0),
    )(xb, wq_sl, k, v, wo_sl)
    return out[None]


# device time: 78276 ns/iter; 1.3781x vs baseline; 1.3781x over previous
import jax
import jax.numpy as jnp
from jax import lax
from jax.experimental import pallas as pl
from jax.experimental.pallas import tpu as pltpu

N_DEV = 4
SQ = 1024
SKV = 1024
HQ_PER = 8
DH = 128
D_MODEL = 1024
SCALE = 0.08838834764831843


def kernel(x, Wq, K_ext, V_ext, Wo):
    my_pos = lax.axis_index("i")

    xb = x[0].astype(jnp.bfloat16)
    wq_sl = lax.dynamic_slice(
        Wq.astype(jnp.bfloat16), (0, my_pos * HQ_PER * DH), (D_MODEL, HQ_PER * DH)
    )
    wo_sl = lax.dynamic_slice(
        Wo.astype(jnp.bfloat16), (my_pos * HQ_PER * DH, 0), (HQ_PER * DH, D_MODEL)
    )
    k = K_ext[0].transpose(1, 0, 2).astype(jnp.bfloat16)
    v = V_ext[0].transpose(1, 0, 2).astype(jnp.bfloat16)

    def body(x_ref, wq_ref, k_ref, v_ref, wo_ref, out_ref,
             partial_ref, comm_ref, send_sems, recv_sems):
        my = lax.axis_index("i")
        left = lax.rem(my + N_DEV - 1, N_DEV)
        right = lax.rem(my + 1, N_DEV)

        q = jnp.dot(x_ref[...], wq_ref[...],
                    preferred_element_type=jnp.float32).astype(jnp.bfloat16)

        rows = lax.broadcasted_iota(jnp.int32, (SQ, SKV), 0)
        cols = lax.broadcasted_iota(jnp.int32, (SQ, SKV), 1)
        mask = ((rows // 64) % 4) == ((cols // 64) % 4)

        ctx_parts = []
        for h in range(HQ_PER):
            q_h = q[:, h * DH:(h + 1) * DH]
            s = lax.dot_general(
                q_h, k_ref[h], (((1,), (1,)), ((), ())),
                preferred_element_type=jnp.float32,
            ) * SCALE
            s = jnp.where(mask, s, -1e9)
            m = jnp.max(s, axis=-1, keepdims=True)
            w = jnp.exp(s - m)
            w = w / jnp.sum(w, axis=-1, keepdims=True)
            ctx_parts.append(
                jnp.dot(w.astype(jnp.bfloat16), v_ref[h],
                        preferred_element_type=jnp.float32).astype(jnp.bfloat16)
            )
        ctx = jnp.concatenate(ctx_parts, axis=1)

        partial_ref[...] = jnp.dot(ctx, wo_ref[...],
                                   preferred_element_type=jnp.float32)

        barrier_sem = pltpu.get_barrier_semaphore()
        for nbr in [left, right]:
            pl.semaphore_signal(
                barrier_sem, inc=1,
                device_id=(nbr,), device_id_type=pl.DeviceIdType.MESH,
            )
        pl.semaphore_wait(barrier_sem, 2)

        CH = SQ // N_DEV

        def hop(send_slot, recv_slot):
            rdma = pltpu.make_async_remote_copy(
                src_ref=comm_ref.at[send_slot],
                dst_ref=comm_ref.at[recv_slot],
                send_sem=send_sems.at[send_slot],
                recv_sem=recv_sems.at[recv_slot],
                device_id=(right,),
                device_id_type=pl.DeviceIdType.MESH,
            )
            rdma.start()
            rdma.wait()

        acc = None
        for s in range(N_DEV - 1):
            send_slot = s % 2
            recv_slot = (s + 1) % 2
            if s == 0:
                c_send = lax.rem(my + N_DEV, N_DEV)
                comm_ref[send_slot, :, :] = partial_ref[
                    pl.ds(c_send * CH, CH), :].astype(jnp.bfloat16)
            else:
                comm_ref[send_slot, :, :] = acc.astype(jnp.bfloat16)
            hop(send_slot, recv_slot)
            c_recv = lax.rem(my - s - 1 + N_DEV, N_DEV)
            acc = (partial_ref[pl.ds(c_recv * CH, CH), :]
                   + comm_ref[recv_slot].astype(jnp.float32))

        c_own = lax.rem(my + 1, N_DEV)
        out_ref[pl.ds(c_own * CH, CH), :] = acc

        for t in range(N_DEV - 1):
            h = N_DEV - 1 + t
            send_slot = h % 2
            recv_slot = (h + 1) % 2
            if t == 0:
                comm_ref[send_slot, :, :] = acc.astype(jnp.bfloat16)
            hop(send_slot, recv_slot)
            c_recv = lax.rem(my - t + N_DEV, N_DEV)
            out_ref[pl.ds(c_recv * CH, CH), :] = comm_ref[
                recv_slot].astype(jnp.float32)

    out = pl.pallas_call(
        body,
        out_shape=jax.ShapeDtypeStruct((SQ, D_MODEL), jnp.float32),
        in_specs=[pl.BlockSpec(memory_space=pltpu.VMEM)] * 5,
        out_specs=pl.BlockSpec(memory_space=pltpu.VMEM),
        scratch_shapes=[
            pltpu.VMEM((SQ, D_MODEL), jnp.float32),
            pltpu.VMEM((2, SQ // N_DEV, D_MODEL), jnp.bfloat16),
            pltpu.SemaphoreType.DMA((2,)),
            pltpu.SemaphoreType.DMA((2,)),
        ],
        compiler_params=pltpu.CompilerParams(collective_id=0),
    )(xb, wq_sl, k, v, wo_sl)
    return out[None]


# device time: 73448 ns/iter; 1.4687x vs baseline; 1.0657x over previous
import jax
import jax.numpy as jnp
from jax import lax
from jax.experimental import pallas as pl
from jax.experimental.pallas import tpu as pltpu

N_DEV = 4
SQ = 1024
SKV = 1024
HQ_PER = 8
DH = 128
D_MODEL = 1024
SCALE = 0.08838834764831843


def kernel(x, Wq, K_ext, V_ext, Wo):
    my_pos = lax.axis_index("i")

    def perm_rows(t):
        s = t.shape
        return t.reshape(*s[:-2], 4, 4, 64, s[-1]).swapaxes(-3, -4).reshape(s)

    xb = perm_rows(x[0].astype(jnp.bfloat16))
    wq_sl = lax.dynamic_slice(
        Wq.astype(jnp.bfloat16), (0, my_pos * HQ_PER * DH), (D_MODEL, HQ_PER * DH)
    )
    wo_sl = lax.dynamic_slice(
        Wo.astype(jnp.bfloat16), (my_pos * HQ_PER * DH, 0), (HQ_PER * DH, D_MODEL)
    )
    k = perm_rows(K_ext[0].transpose(1, 0, 2).astype(jnp.bfloat16))
    v = perm_rows(V_ext[0].transpose(1, 0, 2).astype(jnp.bfloat16))

    def body(x_ref, wq_ref, k_ref, v_ref, wo_ref, out_ref,
             partial_ref, comm_ref, send_sems, recv_sems):
        my = lax.axis_index("i")
        left = lax.rem(my + N_DEV - 1, N_DEV)
        right = lax.rem(my + 1, N_DEV)

        q = jnp.dot(x_ref[...], wq_ref[...],
                    preferred_element_type=jnp.float32).astype(jnp.bfloat16)

        G = SQ // 4
        ctx_parts = []
        for h in range(HQ_PER):
            ctx_gs = []
            for r in range(4):
                q_g = q[r * G:(r + 1) * G, h * DH:(h + 1) * DH]
                k_g = k_ref[h, r * G:(r + 1) * G, :]
                s = lax.dot_general(
                    q_g, k_g, (((1,), (1,)), ((), ())),
                    preferred_element_type=jnp.float32,
                ) * SCALE
                m = jnp.max(s, axis=-1, keepdims=True)
                w = jnp.exp(s - m)
                w = w / jnp.sum(w, axis=-1, keepdims=True)
                ctx_gs.append(
                    jnp.dot(w.astype(jnp.bfloat16),
                            v_ref[h, r * G:(r + 1) * G, :],
                            preferred_element_type=jnp.float32)
                    .astype(jnp.bfloat16)
                )
            ctx_parts.append(jnp.concatenate(ctx_gs, axis=0))
        ctx = jnp.concatenate(ctx_parts, axis=1)

        partial_ref[...] = jnp.dot(ctx, wo_ref[...],
                                   preferred_element_type=jnp.float32)

        barrier_sem = pltpu.get_barrier_semaphore()
        for nbr in [left, right]:
            pl.semaphore_signal(
                barrier_sem, inc=1,
                device_id=(nbr,), device_id_type=pl.DeviceIdType.MESH,
            )
        pl.semaphore_wait(barrier_sem, 2)

        CH = SQ // N_DEV

        def hop(send_slot, recv_slot):
            rdma = pltpu.make_async_remote_copy(
                src_ref=comm_ref.at[send_slot],
                dst_ref=comm_ref.at[recv_slot],
                send_sem=send_sems.at[send_slot],
                recv_sem=recv_sems.at[recv_slot],
                device_id=(right,),
                device_id_type=pl.DeviceIdType.MESH,
            )
            rdma.start()
            rdma.wait()

        acc = None
        for s in range(N_DEV - 1):
            send_slot = s % 2
            recv_slot = (s + 1) % 2
            if s == 0:
                c_send = lax.rem(my + N_DEV, N_DEV)
                comm_ref[send_slot, :, :] = partial_ref[
                    pl.ds(c_send * CH, CH), :].astype(jnp.bfloat16)
            else:
                comm_ref[send_slot, :, :] = acc.astype(jnp.bfloat16)
            hop(send_slot, recv_slot)
            c_recv = lax.rem(my - s - 1 + N_DEV, N_DEV)
            acc = (partial_ref[pl.ds(c_recv * CH, CH), :]
                   + comm_ref[recv_slot].astype(jnp.float32))

        c_own = lax.rem(my + 1, N_DEV)
        out_ref[pl.ds(c_own * CH, CH), :] = acc

        for t in range(N_DEV - 1):
            h = N_DEV - 1 + t
            send_slot = h % 2
            recv_slot = (h + 1) % 2
            if t == 0:
                comm_ref[send_slot, :, :] = acc.astype(jnp.bfloat16)
            hop(send_slot, recv_slot)
            c_recv = lax.rem(my - t + N_DEV, N_DEV)
            out_ref[pl.ds(c_recv * CH, CH), :] = comm_ref[
                recv_slot].astype(jnp.float32)

    out = pl.pallas_call(
        body,
        out_shape=jax.ShapeDtypeStruct((SQ, D_MODEL), jnp.float32),
        in_specs=[pl.BlockSpec(memory_space=pltpu.VMEM)] * 5,
        out_specs=pl.BlockSpec(memory_space=pltpu.VMEM),
        scratch_shapes=[
            pltpu.VMEM((SQ, D_MODEL), jnp.float32),
            pltpu.VMEM((2, SQ // N_DEV, D_MODEL), jnp.bfloat16),
            pltpu.SemaphoreType.DMA((2,)),
            pltpu.SemaphoreType.DMA((2,)),
        ],
        compiler_params=pltpu.CompilerParams(collective_id=0),
    )(xb, wq_sl, k, v, wo_sl)
    return perm_rows(out)[None]


# device time: 56858 ns/iter; 1.8973x vs baseline; 1.2918x over previous
import jax
import jax.numpy as jnp
from jax import lax
from jax.experimental import pallas as pl
from jax.experimental.pallas import tpu as pltpu

N_DEV = 4
SQ = 1024
SKV = 1024
HQ_PER = 8
DH = 128
D_MODEL = 1024
SCALE = 0.08838834764831843


def kernel(x, Wq, K_ext, V_ext, Wo):
    my_pos = lax.axis_index("i")

    def perm_rows(t):
        s = t.shape
        return t.reshape(*s[:-2], 4, 4, 64, s[-1]).swapaxes(-3, -4).reshape(s)

    xb = perm_rows(x[0].astype(jnp.bfloat16))
    wq_sl = lax.dynamic_slice(
        Wq.astype(jnp.bfloat16), (0, my_pos * HQ_PER * DH), (D_MODEL, HQ_PER * DH)
    )
    wo_sl = lax.dynamic_slice(
        Wo.astype(jnp.bfloat16), (my_pos * HQ_PER * DH, 0), (HQ_PER * DH, D_MODEL)
    )
    k = perm_rows(K_ext[0].transpose(1, 0, 2).astype(jnp.bfloat16))
    v = perm_rows(V_ext[0].transpose(1, 0, 2).astype(jnp.bfloat16))

    def body(x_ref, wq_ref, k_ref, v_ref, wo_ref, out_ref,
             partial_ref, rs_send_buf, rs_recv_buf, ag_send_buf, ag_recv_buf,
             rs_send_sems, rs_recv_sems, ag_send_sems, ag_recv_sems):
        my = lax.axis_index("i")

        barrier_sem = pltpu.get_barrier_semaphore()
        for d in range(1, N_DEV):
            pl.semaphore_signal(
                barrier_sem, inc=1,
                device_id=(lax.rem(my + d, N_DEV),),
                device_id_type=pl.DeviceIdType.MESH,
            )
        pl.semaphore_wait(barrier_sem, N_DEV - 1)

        q = jnp.dot(x_ref[...], wq_ref[...],
                    preferred_element_type=jnp.float32).astype(jnp.bfloat16)

        G = SQ // 4
        ctx_parts = []
        for h in range(HQ_PER):
            ctx_gs = []
            for r in range(4):
                q_g = q[r * G:(r + 1) * G, h * DH:(h + 1) * DH]
                k_g = k_ref[h, r * G:(r + 1) * G, :]
                s = lax.dot_general(
                    q_g, k_g, (((1,), (1,)), ((), ())),
                    preferred_element_type=jnp.float32,
                ) * SCALE
                m = jnp.max(s, axis=-1, keepdims=True)
                w = jnp.exp(s - m)
                w = w / jnp.sum(w, axis=-1, keepdims=True)
                ctx_gs.append(
                    jnp.dot(w.astype(jnp.bfloat16),
                            v_ref[h, r * G:(r + 1) * G, :],
                            preferred_element_type=jnp.float32)
                    .astype(jnp.bfloat16)
                )
            ctx_parts.append(jnp.concatenate(ctx_gs, axis=0))
        ctx = jnp.concatenate(ctx_parts, axis=1)

        partial_ref[...] = jnp.dot(ctx, wo_ref[...],
                                   preferred_element_type=jnp.float32)

        CH = SQ // N_DEV

        rs_sends = []
        for d in range(1, N_DEV):
            peer = lax.rem(my + d, N_DEV)
            slot = d - 1
            dst_slot = (N_DEV - d) - 1
            rs_send_buf[slot, :, :] = partial_ref[
                pl.ds(peer * CH, CH), :].astype(jnp.bfloat16)
            send = pltpu.make_async_remote_copy(
                src_ref=rs_send_buf.at[slot],
                dst_ref=rs_recv_buf.at[dst_slot],
                send_sem=rs_send_sems.at[slot],
                recv_sem=rs_recv_sems.at[dst_slot],
                device_id=(peer,),
                device_id_type=pl.DeviceIdType.MESH,
            )
            send.start()
            rs_sends.append(send)

        acc = partial_ref[pl.ds(my * CH, CH), :]
        for j in range(N_DEV - 1):
            recv = pltpu.make_async_remote_copy(
                src_ref=rs_send_buf.at[0],
                dst_ref=rs_recv_buf.at[j],
                send_sem=rs_send_sems.at[0],
                recv_sem=rs_recv_sems.at[j],
                device_id=(my,),
                device_id_type=pl.DeviceIdType.MESH,
            )
            recv.wait_recv()
        for j in range(N_DEV - 1):
            acc = acc + rs_recv_buf[j].astype(jnp.float32)
        out_ref[pl.ds(my * CH, CH), :] = acc

        ag_send_buf[...] = acc.astype(jnp.bfloat16)
        ag_sends = []
        for d in range(1, N_DEV):
            peer = lax.rem(my + d, N_DEV)
            dst_slot = (N_DEV - d) - 1
            send = pltpu.make_async_remote_copy(
                src_ref=ag_send_buf,
                dst_ref=ag_recv_buf.at[dst_slot],
                send_sem=ag_send_sems.at[d - 1],
                recv_sem=ag_recv_sems.at[dst_slot],
                device_id=(peer,),
                device_id_type=pl.DeviceIdType.MESH,
            )
            send.start()
            ag_sends.append(send)

        for j in range(N_DEV - 1):
            recv = pltpu.make_async_remote_copy(
                src_ref=ag_send_buf,
                dst_ref=ag_recv_buf.at[j],
                send_sem=ag_send_sems.at[0],
                recv_sem=ag_recv_sems.at[j],
                device_id=(my,),
                device_id_type=pl.DeviceIdType.MESH,
            )
            recv.wait_recv()
            c = lax.rem(my + j + 1, N_DEV)
            out_ref[pl.ds(c * CH, CH), :] = ag_recv_buf[j].astype(jnp.float32)

        for s in rs_sends + ag_sends:
            s.wait_send()

    out = pl.pallas_call(
        body,
        out_shape=jax.ShapeDtypeStruct((SQ, D_MODEL), jnp.float32),
        in_specs=[pl.BlockSpec(memory_space=pltpu.VMEM)] * 5,
        out_specs=pl.BlockSpec(memory_space=pltpu.VMEM),
        scratch_shapes=[
            pltpu.VMEM((SQ, D_MODEL), jnp.float32),
            pltpu.VMEM((3, SQ // N_DEV, D_MODEL), jnp.bfloat16),
            pltpu.VMEM((3, SQ // N_DEV, D_MODEL), jnp.bfloat16),
            pltpu.VMEM((SQ // N_DEV, D_MODEL), jnp.bfloat16),
            pltpu.VMEM((3, SQ // N_DEV, D_MODEL), jnp.bfloat16),
            pltpu.SemaphoreType.DMA((3,)),
            pltpu.SemaphoreType.DMA((3,)),
            pltpu.SemaphoreType.DMA((3,)),
            pltpu.SemaphoreType.DMA((3,)),
        ],
        compiler_params=pltpu.CompilerParams(collective_id=0),
    )(xb, wq_sl, k, v, wo_sl)
    return perm_rows(out)[None]


# device time: 52675 ns/iter; 2.0479x vs baseline; 1.0794x over previous
import jax
import jax.numpy as jnp
from jax import lax
from jax.experimental import pallas as pl
from jax.experimental.pallas import tpu as pltpu

N_DEV = 4
SQ = 1024
SKV = 1024
HQ_PER = 8
DH = 128
D_MODEL = 1024
SCALE = 0.08838834764831843


def kernel(x, Wq, K_ext, V_ext, Wo):
    my_pos = lax.axis_index("i")

    def perm_rows(t):
        s = t.shape
        return t.reshape(*s[:-2], 4, 4, 64, s[-1]).swapaxes(-3, -4).reshape(s)

    xb = perm_rows(x[0].astype(jnp.bfloat16))
    wq_sl = lax.dynamic_slice(
        Wq.astype(jnp.bfloat16), (0, my_pos * HQ_PER * DH), (D_MODEL, HQ_PER * DH)
    )
    wo_sl = lax.dynamic_slice(
        Wo.astype(jnp.bfloat16), (my_pos * HQ_PER * DH, 0), (HQ_PER * DH, D_MODEL)
    )
    k = perm_rows(K_ext[0].transpose(1, 0, 2).astype(jnp.bfloat16))
    v = perm_rows(V_ext[0].transpose(1, 0, 2).astype(jnp.bfloat16))

    def body(x_ref, wq_ref, k_ref, v_ref, wo_ref, out_ref,
             rs_send_buf, rs_recv_buf, ag_send_buf, ag_recv_buf,
             rs_send_sems, rs_recv_sems, ag_send_sems, ag_recv_sems):
        my = lax.axis_index("i")

        barrier_sem = pltpu.get_barrier_semaphore()
        for d in range(1, N_DEV):
            pl.semaphore_signal(
                barrier_sem, inc=1,
                device_id=(lax.rem(my + d, N_DEV),),
                device_id_type=pl.DeviceIdType.MESH,
            )
        pl.semaphore_wait(barrier_sem, N_DEV - 1)

        CH = SQ // N_DEV

        def compute_chunk(c):
            xq = x_ref[pl.ds(c * CH, CH), :]
            qc = jnp.dot(xq, wq_ref[...],
                         preferred_element_type=jnp.float32).astype(jnp.bfloat16)
            ctx_hs = []
            for h in range(HQ_PER):
                q_h = qc[:, h * DH:(h + 1) * DH]
                k_g = k_ref[h, pl.ds(c * CH, CH), :]
                v_g = v_ref[h, pl.ds(c * CH, CH), :]
                s = lax.dot_general(
                    q_h, k_g, (((1,), (1,)), ((), ())),
                    preferred_element_type=jnp.float32,
                ) * SCALE
                m = jnp.max(s, axis=-1, keepdims=True)
                w = jnp.exp(s - m)
                w = w / jnp.sum(w, axis=-1, keepdims=True)
                ctx_hs.append(
                    jnp.dot(w.astype(jnp.bfloat16), v_g,
                            preferred_element_type=jnp.float32)
                    .astype(jnp.bfloat16)
                )
            ctxc = jnp.concatenate(ctx_hs, axis=1)
            return jnp.dot(ctxc, wo_ref[...],
                           preferred_element_type=jnp.float32)

        rs_sends = []
        for d in range(1, N_DEV):
            peer = lax.rem(my + d, N_DEV)
            slot = d - 1
            dst_slot = (N_DEV - d) - 1
            rs_send_buf[slot, :, :] = compute_chunk(peer).astype(jnp.bfloat16)
            send = pltpu.make_async_remote_copy(
                src_ref=rs_send_buf.at[slot],
                dst_ref=rs_recv_buf.at[dst_slot],
                send_sem=rs_send_sems.at[slot],
                recv_sem=rs_recv_sems.at[dst_slot],
                device_id=(peer,),
                device_id_type=pl.DeviceIdType.MESH,
            )
            send.start()
            rs_sends.append(send)

        acc = compute_chunk(my)
        for j in range(N_DEV - 1):
            recv = pltpu.make_async_remote_copy(
                src_ref=rs_send_buf.at[0],
                dst_ref=rs_recv_buf.at[j],
                send_sem=rs_send_sems.at[0],
                recv_sem=rs_recv_sems.at[j],
                device_id=(my,),
                device_id_type=pl.DeviceIdType.MESH,
            )
            recv.wait_recv()
        for j in range(N_DEV - 1):
            acc = acc + rs_recv_buf[j].astype(jnp.float32)

        ag_send_buf[...] = acc.astype(jnp.bfloat16)
        ag_sends = []
        for d in range(1, N_DEV):
            peer = lax.rem(my + d, N_DEV)
            dst_slot = (N_DEV - d) - 1
            send = pltpu.make_async_remote_copy(
                src_ref=ag_send_buf,
                dst_ref=ag_recv_buf.at[dst_slot],
                send_sem=ag_send_sems.at[d - 1],
                recv_sem=ag_recv_sems.at[dst_slot],
                device_id=(peer,),
                device_id_type=pl.DeviceIdType.MESH,
            )
            send.start()
            ag_sends.append(send)

        out_ref[pl.ds(my * CH, CH), :] = acc

        for j in range(N_DEV - 1):
            recv = pltpu.make_async_remote_copy(
                src_ref=ag_send_buf,
                dst_ref=ag_recv_buf.at[j],
                send_sem=ag_send_sems.at[0],
                recv_sem=ag_recv_sems.at[j],
                device_id=(my,),
                device_id_type=pl.DeviceIdType.MESH,
            )
            recv.wait_recv()
            c = lax.rem(my + j + 1, N_DEV)
            out_ref[pl.ds(c * CH, CH), :] = ag_recv_buf[j].astype(jnp.float32)

        for s in rs_sends + ag_sends:
            s.wait_send()

    out = pl.pallas_call(
        body,
        out_shape=jax.ShapeDtypeStruct((SQ, D_MODEL), jnp.float32),
        in_specs=[pl.BlockSpec(memory_space=pltpu.VMEM)] * 5,
        out_specs=pl.BlockSpec(memory_space=pltpu.VMEM),
        scratch_shapes=[
            pltpu.VMEM((3, SQ // N_DEV, D_MODEL), jnp.bfloat16),
            pltpu.VMEM((3, SQ // N_DEV, D_MODEL), jnp.bfloat16),
            pltpu.VMEM((SQ // N_DEV, D_MODEL), jnp.bfloat16),
            pltpu.VMEM((3, SQ // N_DEV, D_MODEL), jnp.bfloat16),
            pltpu.SemaphoreType.DMA((3,)),
            pltpu.SemaphoreType.DMA((3,)),
            pltpu.SemaphoreType.DMA((3,)),
            pltpu.SemaphoreType.DMA((3,)),
        ],
        compiler_params=pltpu.CompilerParams(collective_id=0),
    )(xb, wq_sl, k, v, wo_sl)
    return perm_rows(out)[None]


# device time: 52137 ns/iter; 2.0691x vs baseline; 1.0103x over previous
import jax
import jax.numpy as jnp
from jax import lax
from jax.experimental import pallas as pl
from jax.experimental.pallas import tpu as pltpu

N_DEV = 4
SQ = 1024
SKV = 1024
HQ_PER = 8
DH = 128
D_MODEL = 1024
SCALE = 0.08838834764831843


def kernel(x, Wq, K_ext, V_ext, Wo):
    my_pos = lax.axis_index("i")

    def perm_rows(t):
        s = t.shape
        return t.reshape(*s[:-2], 4, 4, 64, s[-1]).swapaxes(-3, -4).reshape(s)

    xb = x[0].astype(jnp.bfloat16)
    wq_sl = lax.dynamic_slice(
        Wq.astype(jnp.bfloat16), (0, my_pos * HQ_PER * DH), (D_MODEL, HQ_PER * DH)
    )
    wo_sl = lax.dynamic_slice(
        Wo.astype(jnp.bfloat16), (my_pos * HQ_PER * DH, 0), (HQ_PER * DH, D_MODEL)
    )
    k = perm_rows(K_ext[0].transpose(1, 0, 2).astype(jnp.bfloat16))
    v = perm_rows(V_ext[0].transpose(1, 0, 2).astype(jnp.bfloat16))

    def body(x_ref, wq_ref, k_ref, v_ref, wo_ref, out_ref,
             rs_send_buf, rs_recv_buf, ag_send_buf, ag_recv_buf,
             rs_send_sems, rs_recv_sems, ag_send_sems, ag_recv_sems):
        my = lax.axis_index("i")

        barrier_sem = pltpu.get_barrier_semaphore()
        for d in range(1, N_DEV):
            pl.semaphore_signal(
                barrier_sem, inc=1,
                device_id=(lax.rem(my + d, N_DEV),),
                device_id_type=pl.DeviceIdType.MESH,
            )
        pl.semaphore_wait(barrier_sem, N_DEV - 1)

        CH = SQ // N_DEV

        def compute_chunk(c):
            xq = jnp.concatenate(
                [x_ref[pl.ds((c + 4 * j) * 64, 64), :] for j in range(4)],
                axis=0)
            qc = jnp.dot(xq, wq_ref[...],
                         preferred_element_type=jnp.float32).astype(jnp.bfloat16)
            ctx_hs = []
            for h in range(HQ_PER):
                q_h = qc[:, h * DH:(h + 1) * DH]
                k_g = k_ref[h, pl.ds(c * CH, CH), :]
                v_g = v_ref[h, pl.ds(c * CH, CH), :]
                s = lax.dot_general(
                    q_h, k_g, (((1,), (1,)), ((), ())),
                    preferred_element_type=jnp.float32,
                ) * SCALE
                m = jnp.max(s, axis=-1, keepdims=True)
                w = jnp.exp(s - m)
                w = w / jnp.sum(w, axis=-1, keepdims=True)
                ctx_hs.append(
                    jnp.dot(w.astype(jnp.bfloat16), v_g,
                            preferred_element_type=jnp.float32)
                    .astype(jnp.bfloat16)
                )
            ctxc = jnp.concatenate(ctx_hs, axis=1)
            return jnp.dot(ctxc, wo_ref[...],
                           preferred_element_type=jnp.float32)

        rs_sends = []
        for d in range(1, N_DEV):
            peer = lax.rem(my + d, N_DEV)
            slot = d - 1
            dst_slot = (N_DEV - d) - 1
            rs_send_buf[slot, :, :] = compute_chunk(peer).astype(jnp.bfloat16)
            send = pltpu.make_async_remote_copy(
                src_ref=rs_send_buf.at[slot],
                dst_ref=rs_recv_buf.at[dst_slot],
                send_sem=rs_send_sems.at[slot],
                recv_sem=rs_recv_sems.at[dst_slot],
                device_id=(peer,),
                device_id_type=pl.DeviceIdType.MESH,
            )
            send.start()
            rs_sends.append(send)

        def store_chunk(c, val):
            for j in range(4):
                out_ref[pl.ds((c + 4 * j) * 64, 64), :] = val[
                    j * 64:(j + 1) * 64, :]

        acc = compute_chunk(my)
        for j in range(N_DEV - 1):
            recv = pltpu.make_async_remote_copy(
                src_ref=rs_send_buf.at[0],
                dst_ref=rs_recv_buf.at[j],
                send_sem=rs_send_sems.at[0],
                recv_sem=rs_recv_sems.at[j],
                device_id=(my,),
                device_id_type=pl.DeviceIdType.MESH,
            )
            recv.wait_recv()
        for j in range(N_DEV - 1):
            acc = acc + rs_recv_buf[j].astype(jnp.float32)

        ag_send_buf[...] = acc.astype(jnp.bfloat16)
        ag_sends = []
        for d in range(1, N_DEV):
            peer = lax.rem(my + d, N_DEV)
            dst_slot = (N_DEV - d) - 1
            send = pltpu.make_async_remote_copy(
                src_ref=ag_send_buf,
                dst_ref=ag_recv_buf.at[dst_slot],
                send_sem=ag_send_sems.at[d - 1],
                recv_sem=ag_recv_sems.at[dst_slot],
                device_id=(peer,),
                device_id_type=pl.DeviceIdType.MESH,
            )
            send.start()
            ag_sends.append(send)

        store_chunk(my, acc)

        for j in range(N_DEV - 1):
            recv = pltpu.make_async_remote_copy(
                src_ref=ag_send_buf,
                dst_ref=ag_recv_buf.at[j],
                send_sem=ag_send_sems.at[0],
                recv_sem=ag_recv_sems.at[j],
                device_id=(my,),
                device_id_type=pl.DeviceIdType.MESH,
            )
            recv.wait_recv()
            c = lax.rem(my + j + 1, N_DEV)
            store_chunk(c, ag_recv_buf[j].astype(jnp.float32))

        for s in rs_sends + ag_sends:
            s.wait_send()

    out = pl.pallas_call(
        body,
        out_shape=jax.ShapeDtypeStruct((SQ, D_MODEL), jnp.float32),
        in_specs=[pl.BlockSpec(memory_space=pltpu.VMEM)] * 5,
        out_specs=pl.BlockSpec(memory_space=pltpu.VMEM),
        scratch_shapes=[
            pltpu.VMEM((3, SQ // N_DEV, D_MODEL), jnp.bfloat16),
            pltpu.VMEM((3, SQ // N_DEV, D_MODEL), jnp.bfloat16),
            pltpu.VMEM((SQ // N_DEV, D_MODEL), jnp.bfloat16),
            pltpu.VMEM((3, SQ // N_DEV, D_MODEL), jnp.bfloat16),
            pltpu.SemaphoreType.DMA((3,)),
            pltpu.SemaphoreType.DMA((3,)),
            pltpu.SemaphoreType.DMA((3,)),
            pltpu.SemaphoreType.DMA((3,)),
        ],
        compiler_params=pltpu.CompilerParams(collective_id=0),
    )(xb, wq_sl, k, v, wo_sl)
    return out[None]


# device time: 45615 ns/iter; 2.3649x vs baseline; 1.1430x over previous
import jax
import jax.numpy as jnp
from jax import lax
from jax.experimental import pallas as pl
from jax.experimental.pallas import tpu as pltpu

N_DEV = 4
SQ = 1024
SKV = 1024
HQ_PER = 8
DH = 128
D_MODEL = 1024
SCALE = 0.08838834764831843


def kernel(x, Wq, K_ext, V_ext, Wo):
    my_pos = lax.axis_index("i")

    xb = x[0].astype(jnp.bfloat16)
    wq_sl = lax.dynamic_slice(
        Wq.astype(jnp.bfloat16), (0, my_pos * HQ_PER * DH), (D_MODEL, HQ_PER * DH)
    )
    wo_sl = lax.dynamic_slice(
        Wo.astype(jnp.bfloat16), (my_pos * HQ_PER * DH, 0), (HQ_PER * DH, D_MODEL)
    )
    k = K_ext.reshape(SKV, HQ_PER * DH).astype(jnp.bfloat16)
    v = V_ext.reshape(SKV, HQ_PER * DH).astype(jnp.bfloat16)

    def body(x_ref, wq_ref, k_ref, v_ref, wo_ref, out_ref,
             rs_send_buf, rs_recv_buf, ag_send_buf, ag_recv_buf,
             rs_send_sems, rs_recv_sems, ag_send_sems, ag_recv_sems):
        my = lax.axis_index("i")

        barrier_sem = pltpu.get_barrier_semaphore()
        for d in range(1, N_DEV):
            pl.semaphore_signal(
                barrier_sem, inc=1,
                device_id=(lax.rem(my + d, N_DEV),),
                device_id_type=pl.DeviceIdType.MESH,
            )
        pl.semaphore_wait(barrier_sem, N_DEV - 1)

        CH = SQ // N_DEV

        def gather_group(ref, c):
            return jnp.concatenate(
                [ref[pl.ds((c + 4 * j) * 64, 64), :] for j in range(4)],
                axis=0)

        def compute_chunk(c):
            xq = gather_group(x_ref, c)
            qc = jnp.dot(xq, wq_ref[...],
                         preferred_element_type=jnp.float32).astype(jnp.bfloat16)
            kc = gather_group(k_ref, c)
            vc = gather_group(v_ref, c)
            ctx_hs = []
            for h in range(HQ_PER):
                q_h = qc[:, h * DH:(h + 1) * DH]
                k_g = kc[:, h * DH:(h + 1) * DH]
                v_g = vc[:, h * DH:(h + 1) * DH]
                s = lax.dot_general(
                    q_h, k_g, (((1,), (1,)), ((), ())),
                    preferred_element_type=jnp.float32,
                ) * SCALE
                m = jnp.max(s, axis=-1, keepdims=True)
                w = jnp.exp(s - m)
                w = w / jnp.sum(w, axis=-1, keepdims=True)
                ctx_hs.append(
                    jnp.dot(w.astype(jnp.bfloat16), v_g,
                            preferred_element_type=jnp.float32)
                    .astype(jnp.bfloat16)
                )
            ctxc = jnp.concatenate(ctx_hs, axis=1)
            return jnp.dot(ctxc, wo_ref[...],
                           preferred_element_type=jnp.float32)

        rs_sends = []
        for d in range(1, N_DEV):
            peer = lax.rem(my + d, N_DEV)
            slot = d - 1
            dst_slot = (N_DEV - d) - 1
            rs_send_buf[slot, :, :] = compute_chunk(peer).astype(jnp.bfloat16)
            send = pltpu.make_async_remote_copy(
                src_ref=rs_send_buf.at[slot],
                dst_ref=rs_recv_buf.at[dst_slot],
                send_sem=rs_send_sems.at[slot],
                recv_sem=rs_recv_sems.at[dst_slot],
                device_id=(peer,),
                device_id_type=pl.DeviceIdType.MESH,
            )
            send.start()
            rs_sends.append(send)

        def store_chunk(c, val):
            for j in range(4):
                out_ref[pl.ds((c + 4 * j) * 64, 64), :] = val[
                    j * 64:(j + 1) * 64, :]

        acc = compute_chunk(my)
        for j in range(N_DEV - 1):
            recv = pltpu.make_async_remote_copy(
                src_ref=rs_send_buf.at[0],
                dst_ref=rs_recv_buf.at[j],
                send_sem=rs_send_sems.at[0],
                recv_sem=rs_recv_sems.at[j],
                device_id=(my,),
                device_id_type=pl.DeviceIdType.MESH,
            )
            recv.wait_recv()
        for j in range(N_DEV - 1):
            acc = acc + rs_recv_buf[j].astype(jnp.float32)

        acc = acc.astype(jnp.bfloat16)
        ag_send_buf[...] = acc
        ag_sends = []
        for d in range(1, N_DEV):
            peer = lax.rem(my + d, N_DEV)
            dst_slot = (N_DEV - d) - 1
            send = pltpu.make_async_remote_copy(
                src_ref=ag_send_buf,
                dst_ref=ag_recv_buf.at[dst_slot],
                send_sem=ag_send_sems.at[d - 1],
                recv_sem=ag_recv_sems.at[dst_slot],
                device_id=(peer,),
                device_id_type=pl.DeviceIdType.MESH,
            )
            send.start()
            ag_sends.append(send)

        store_chunk(my, acc)

        for j in range(N_DEV - 1):
            recv = pltpu.make_async_remote_copy(
                src_ref=ag_send_buf,
                dst_ref=ag_recv_buf.at[j],
                send_sem=ag_send_sems.at[0],
                recv_sem=ag_recv_sems.at[j],
                device_id=(my,),
                device_id_type=pl.DeviceIdType.MESH,
            )
            recv.wait_recv()
            c = lax.rem(my + j + 1, N_DEV)
            store_chunk(c, ag_recv_buf[j])

        for s in rs_sends + ag_sends:
            s.wait_send()

    out = pl.pallas_call(
        body,
        out_shape=jax.ShapeDtypeStruct((SQ, D_MODEL), jnp.bfloat16),
        in_specs=[pl.BlockSpec(memory_space=pltpu.VMEM)] * 5,
        out_specs=pl.BlockSpec(memory_space=pltpu.VMEM),
        scratch_shapes=[
            pltpu.VMEM((3, SQ // N_DEV, D_MODEL), jnp.bfloat16),
            pltpu.VMEM((3, SQ // N_DEV, D_MODEL), jnp.bfloat16),
            pltpu.VMEM((SQ // N_DEV, D_MODEL), jnp.bfloat16),
            pltpu.VMEM((3, SQ // N_DEV, D_MODEL), jnp.bfloat16),
            pltpu.SemaphoreType.DMA((3,)),
            pltpu.SemaphoreType.DMA((3,)),
            pltpu.SemaphoreType.DMA((3,)),
            pltpu.SemaphoreType.DMA((3,)),
        ],
        compiler_params=pltpu.CompilerParams(collective_id=0),
    )(xb, wq_sl, k, v, wo_sl)
    return out[None]


# device time: 45574 ns/iter; 2.3670x vs baseline; 1.0009x over previous
import jax
import jax.numpy as jnp
from jax import lax
from jax.experimental import pallas as pl
from jax.experimental.pallas import tpu as pltpu

N_DEV = 4
SQ = 1024
SKV = 1024
HQ_PER = 8
DH = 128
D_MODEL = 1024
SCALE = 0.08838834764831843


def kernel(x, Wq, K_ext, V_ext, Wo):
    my_pos = lax.axis_index("i")

    xb = x[0]
    wq_sl = lax.dynamic_slice(
        Wq.astype(jnp.bfloat16), (0, my_pos * HQ_PER * DH), (D_MODEL, HQ_PER * DH)
    )
    wo_sl = lax.dynamic_slice(
        Wo.astype(jnp.bfloat16), (my_pos * HQ_PER * DH, 0), (HQ_PER * DH, D_MODEL)
    )
    k = K_ext.reshape(SKV, HQ_PER * DH)
    v = V_ext.reshape(SKV, HQ_PER * DH)

    def body(x_ref, wq_ref, k_ref, v_ref, wo_ref, out_ref,
             rs_send_buf, rs_recv_buf, ag_send_buf, ag_recv_buf,
             rs_send_sems, rs_recv_sems, ag_send_sems, ag_recv_sems):
        my = lax.axis_index("i")

        barrier_sem = pltpu.get_barrier_semaphore()
        for d in range(1, N_DEV):
            pl.semaphore_signal(
                barrier_sem, inc=1,
                device_id=(lax.rem(my + d, N_DEV),),
                device_id_type=pl.DeviceIdType.MESH,
            )
        pl.semaphore_wait(barrier_sem, N_DEV - 1)

        CH = SQ // N_DEV

        def gather_group(ref, c):
            return jnp.concatenate(
                [ref[pl.ds((c + 4 * j) * 64, 64), :] for j in range(4)],
                axis=0).astype(jnp.bfloat16)

        def compute_chunk(c):
            xq = gather_group(x_ref, c)
            qc = jnp.dot(xq, wq_ref[...],
                         preferred_element_type=jnp.float32).astype(jnp.bfloat16)
            kc = gather_group(k_ref, c)
            vc = gather_group(v_ref, c)
            ctx_hs = []
            for h in range(HQ_PER):
                q_h = qc[:, h * DH:(h + 1) * DH]
                k_g = kc[:, h * DH:(h + 1) * DH]
                v_g = vc[:, h * DH:(h + 1) * DH]
                s = lax.dot_general(
                    q_h, k_g, (((1,), (1,)), ((), ())),
                    preferred_element_type=jnp.float32,
                ) * SCALE
                m = jnp.max(s, axis=-1, keepdims=True)
                w = jnp.exp(s - m)
                w = w / jnp.sum(w, axis=-1, keepdims=True)
                ctx_hs.append(
                    jnp.dot(w.astype(jnp.bfloat16), v_g,
                            preferred_element_type=jnp.float32)
                    .astype(jnp.bfloat16)
                )
            ctxc = jnp.concatenate(ctx_hs, axis=1)
            return jnp.dot(ctxc, wo_ref[...],
                           preferred_element_type=jnp.float32)

        rs_sends = []
        for d in range(1, N_DEV):
            peer = lax.rem(my + d, N_DEV)
            slot = d - 1
            dst_slot = (N_DEV - d) - 1
            rs_send_buf[slot, :, :] = compute_chunk(peer).astype(jnp.bfloat16)
            send = pltpu.make_async_remote_copy(
                src_ref=rs_send_buf.at[slot],
                dst_ref=rs_recv_buf.at[dst_slot],
                send_sem=rs_send_sems.at[slot],
                recv_sem=rs_recv_sems.at[dst_slot],
                device_id=(peer,),
                device_id_type=pl.DeviceIdType.MESH,
            )
            send.start()
            rs_sends.append(send)

        def store_chunk(c, val):
            for j in range(4):
                out_ref[pl.ds((c + 4 * j) * 64, 64), :] = val[
                    j * 64:(j + 1) * 64, :]

        acc = compute_chunk(my)
        for j in range(N_DEV - 1):
            recv = pltpu.make_async_remote_copy(
                src_ref=rs_send_buf.at[0],
                dst_ref=rs_recv_buf.at[j],
                send_sem=rs_send_sems.at[0],
                recv_sem=rs_recv_sems.at[j],
                device_id=(my,),
                device_id_type=pl.DeviceIdType.MESH,
            )
            recv.wait_recv()
        for j in range(N_DEV - 1):
            acc = acc + rs_recv_buf[j].astype(jnp.float32)

        acc = acc.astype(jnp.bfloat16)
        ag_send_buf[...] = acc
        ag_sends = []
        for d in range(1, N_DEV):
            peer = lax.rem(my + d, N_DEV)
            dst_slot = (N_DEV - d) - 1
            send = pltpu.make_async_remote_copy(
                src_ref=ag_send_buf,
                dst_ref=ag_recv_buf.at[dst_slot],
                send_sem=ag_send_sems.at[d - 1],
                recv_sem=ag_recv_sems.at[dst_slot],
                device_id=(peer,),
                device_id_type=pl.DeviceIdType.MESH,
            )
            send.start()
            ag_sends.append(send)

        store_chunk(my, acc)

        for j in range(N_DEV - 1):
            recv = pltpu.make_async_remote_copy(
                src_ref=ag_send_buf,
                dst_ref=ag_recv_buf.at[j],
                send_sem=ag_send_sems.at[0],
                recv_sem=ag_recv_sems.at[j],
                device_id=(my,),
                device_id_type=pl.DeviceIdType.MESH,
            )
            recv.wait_recv()
            c = lax.rem(my + j + 1, N_DEV)
            store_chunk(c, ag_recv_buf[j])

        for s in rs_sends + ag_sends:
            s.wait_send()

    out = pl.pallas_call(
        body,
        out_shape=jax.ShapeDtypeStruct((SQ, D_MODEL), jnp.bfloat16),
        in_specs=[pl.BlockSpec(memory_space=pltpu.VMEM)] * 5,
        out_specs=pl.BlockSpec(memory_space=pltpu.VMEM),
        scratch_shapes=[
            pltpu.VMEM((3, SQ // N_DEV, D_MODEL), jnp.bfloat16),
            pltpu.VMEM((3, SQ // N_DEV, D_MODEL), jnp.bfloat16),
            pltpu.VMEM((SQ // N_DEV, D_MODEL), jnp.bfloat16),
            pltpu.VMEM((3, SQ // N_DEV, D_MODEL), jnp.bfloat16),
            pltpu.SemaphoreType.DMA((3,)),
            pltpu.SemaphoreType.DMA((3,)),
            pltpu.SemaphoreType.DMA((3,)),
            pltpu.SemaphoreType.DMA((3,)),
        ],
        compiler_params=pltpu.CompilerParams(collective_id=0),
    )(xb, wq_sl, k, v, wo_sl)
    return out[None]
